# Initial kernel scaffold; baseline (speedup 1.0000x reference)
import jax
import jax.numpy as jnp
from jax.experimental import pallas as pl


def kernel(x_feat, cluster_labels, edge_index, batch, W0, B0, W1, B1, Wm1, bm1, gamma, beta, Wm2, bm2):
    raise NotImplementedError("write your pallas kernel here")


# speedup vs baseline: 23.8069x; 23.8069x over previous
import functools

import jax
import jax.numpy as jnp
from jax import lax
from jax.experimental import pallas as pl
from jax.experimental.pallas import tpu as pltpu
from jax.experimental.pallas import tpu_sc as plsc

N = 10000
E = 320000
IN = 128
H = 256
OUT = 128
C = 4
G = 64

NC = 2
NS = 16
B_EDGE = 128
STEPS_AGG = 160
STEPS_DEG = 80
CH = 16
E_PAD = NS * STEPS_AGG * B_EDGE
N_SH = 10112
RPT = N_SH // NS

BN = 400
GRID = N // BN


def _zero_fill(buf, rows, width):
    def body(i, _):
        for k in range(width // 16):
            buf[i, pl.ds(16 * k, 16)] = jnp.zeros((16,), jnp.float32)
        return 0
    lax.fori_loop(0, rows, body, 0, unroll=False)


def _copy_zero_rows(buf, dst, base):
    off = 0
    for sz in (128, 128, 128, 128, RPT - 512):
        assert sz > 0
        pltpu.sync_copy(buf.at[pl.ds(0, sz)], dst.at[pl.ds(base + off, sz)])
        off += sz


def _make_deg_kernel():
    mesh = plsc.VectorSubcoreMesh(core_axis_name="c", subcore_axis_name="s")

    @functools.partial(
        pl.kernel, mesh=mesh,
        out_type=(jax.ShapeDtypeStruct((N_SH, 16), jnp.float32),
                  jax.ShapeDtypeStruct((N_SH, 16), jnp.float32)),
        scratch_types=[
            pltpu.VMEM((STEPS_DEG, B_EDGE), jnp.int32),
            pltpu.VMEM((B_EDGE, 16), jnp.float32),
            pltpu.VMEM_SHARED((N_SH, 16), jnp.float32),
        ],
    )
    def deg_kernel(col3d, dega, degb, colv, buf, shd):
        cid = lax.axis_index("c")
        sid = lax.axis_index("s")
        wid = cid * NS + sid
        pltpu.sync_copy(col3d.at[wid], colv)
        _zero_fill(buf, B_EDGE, 16)
        _copy_zero_rows(buf, shd, sid * RPT)

        def fill_ones(i, _):
            buf[i, pl.ds(0, 16)] = jnp.ones((16,), jnp.float32)
            return 0
        lax.fori_loop(0, B_EDGE, fill_ones, 0, unroll=False)

        plsc.subcore_barrier()

        def body(j, _):
            pltpu.sync_copy(buf, shd.at[colv.at[j]], add=True)
            return 0
        lax.fori_loop(0, STEPS_DEG, body, 0, unroll=False)

        plsc.subcore_barrier()

        @pl.when(cid == 0)
        def _():
            pltpu.sync_copy(shd.at[pl.ds(sid * RPT, RPT)],
                            dega.at[pl.ds(sid * RPT, RPT)])

        @pl.when(cid == 1)
        def _():
            pltpu.sync_copy(shd.at[pl.ds(sid * RPT, RPT)],
                            degb.at[pl.ds(sid * RPT, RPT)])

    return deg_kernel


def _make_agg0_kernel():
    mesh = plsc.VectorSubcoreMesh(core_axis_name="c", subcore_axis_name="s")

    @functools.partial(
        pl.kernel, mesh=mesh,
        out_type=(jax.ShapeDtypeStruct((N_SH, IN), jnp.float32),
                  jax.ShapeDtypeStruct((N_SH, IN), jnp.float32)),
        scratch_types=[
            pltpu.VMEM((CH, B_EDGE), jnp.int32),
            pltpu.VMEM((CH, B_EDGE), jnp.int32),
            pltpu.VMEM((B_EDGE, IN), jnp.float32),
            pltpu.VMEM_SHARED((N_SH, IN), jnp.float32),
            pltpu.SemaphoreType.DMA,
        ],
    )
    def agg0_kernel(row3d, col3d, y0, za, zb, rowv, colv, rbuf, shz, sem):
        cid = lax.axis_index("c")
        sid = lax.axis_index("s")
        wid = cid * NS + sid
        _zero_fill(rbuf, B_EDGE, IN)
        _copy_zero_rows(rbuf, shz, sid * RPT)
        plsc.subcore_barrier()

        def chunk(ci, _):
            pltpu.sync_copy(row3d.at[wid, pl.ds(ci * CH, CH)], rowv)
            pltpu.sync_copy(col3d.at[wid, pl.ds(ci * CH, CH)], colv)

            def body(j, _):
                pltpu.async_copy(y0.at[rowv.at[j]], rbuf, sem).wait()
                pltpu.sync_copy(rbuf, shz.at[colv.at[j]], add=True)
                return 0
            lax.fori_loop(0, CH, body, 0, unroll=False)
            return 0
        lax.fori_loop(0, STEPS_DEG // CH, chunk, 0, unroll=False)

        plsc.subcore_barrier()

        @pl.when(cid == 0)
        def _():
            pltpu.sync_copy(shz.at[pl.ds(sid * RPT, RPT)],
                            za.at[pl.ds(sid * RPT, RPT)])

        @pl.when(cid == 1)
        def _():
            pltpu.sync_copy(shz.at[pl.ds(sid * RPT, RPT)],
                            zb.at[pl.ds(sid * RPT, RPT)])

    return agg0_kernel


def _make_agg1_kernel():
    fc = H // 2
    mesh = plsc.VectorSubcoreMesh(core_axis_name="c", subcore_axis_name="s")

    @functools.partial(
        pl.kernel, mesh=mesh,
        out_type=(jax.ShapeDtypeStruct((N_SH, fc), jnp.float32),
                  jax.ShapeDtypeStruct((N_SH, fc), jnp.float32)),
        scratch_types=[
            pltpu.VMEM((CH, B_EDGE), jnp.int32),
            pltpu.VMEM((CH, B_EDGE), jnp.int32),
            pltpu.VMEM((B_EDGE, fc), jnp.float32),
            pltpu.VMEM_SHARED((N_SH, fc), jnp.float32),
            pltpu.SemaphoreType.DMA,
        ],
    )
    def agg1_kernel(row3d, col3d, ya, yb, za, zb, rowv, colv, rbuf, shz, sem):
        cid = lax.axis_index("c")
        sid = lax.axis_index("s")
        _zero_fill(rbuf, B_EDGE, fc)
        _copy_zero_rows(rbuf, shz, sid * RPT)
        plsc.subcore_barrier()

        def run(y_hbm):
            def chunk(ci, _):
                pltpu.sync_copy(row3d.at[sid, pl.ds(ci * CH, CH)], rowv)
                pltpu.sync_copy(col3d.at[sid, pl.ds(ci * CH, CH)], colv)

                def body(j, _):
                    pltpu.async_copy(y_hbm.at[rowv.at[j]], rbuf, sem).wait()
                    pltpu.sync_copy(rbuf, shz.at[colv.at[j]], add=True)
                    return 0
                lax.fori_loop(0, CH, body, 0, unroll=False)
                return 0
            lax.fori_loop(0, STEPS_AGG // CH, chunk, 0, unroll=False)

        @pl.when(cid == 0)
        def _():
            run(ya)

        @pl.when(cid == 1)
        def _():
            run(yb)

        plsc.subcore_barrier()

        @pl.when(cid == 0)
        def _():
            pltpu.sync_copy(shz.at[pl.ds(sid * RPT, RPT)],
                            za.at[pl.ds(sid * RPT, RPT)])

        @pl.when(cid == 1)
        def _():
            pltpu.sync_copy(shz.at[pl.ds(sid * RPT, RPT)],
                            zb.at[pl.ds(sid * RPT, RPT)])

    return agg1_kernel


def _dis_block(dega_ref, degb_ref):
    deg = dega_ref[:, 0:1] + degb_ref[:, 0:1] + 1.0
    return lax.rsqrt(deg)


def _scale_body(dega_ref, degb_ref, x_ref, y_ref):
    dis = _dis_block(dega_ref, degb_ref)
    y_ref[...] = x_ref[...] * dis


def _scale_call(dega, degb, x_feat):
    return pl.pallas_call(
        _scale_body,
        grid=(GRID,),
        in_specs=[
            pl.BlockSpec((BN, 16), lambda i: (i, 0)),
            pl.BlockSpec((BN, 16), lambda i: (i, 0)),
            pl.BlockSpec((BN, IN), lambda i: (i, 0)),
        ],
        out_specs=pl.BlockSpec((BN, IN), lambda i: (i, 0)),
        out_shape=jax.ShapeDtypeStruct((N_SH, IN), jnp.float32),
    )(dega, degb, x_feat)


def _layer0_body(za_ref, zb_ref, y0_ref, dega_ref, degb_ref, lab_ref,
                 w_ref, b_ref, y1a_ref, y1b_ref):
    dis = _dis_block(dega_ref, degb_ref)
    agg = (za_ref[...] + zb_ref[...] + y0_ref[...]) * dis
    lab = lab_ref[...]
    b = b_ref[...]
    x1 = jnp.zeros((BN, H), jnp.float32)
    for j in range(C):
        m = (lab == j).astype(jnp.float32)
        x1 = x1 + m * (jnp.dot(agg, w_ref[j],
                               preferred_element_type=jnp.float32) + b[j][None, :])
    y1 = x1 * dis
    y1a_ref[...] = y1[:, :H // 2]
    y1b_ref[...] = y1[:, H // 2:]


def _layer0_call(za, zb, y0, dega, degb, lab2d, W0, B0):
    return pl.pallas_call(
        _layer0_body,
        grid=(GRID,),
        in_specs=[
            pl.BlockSpec((BN, IN), lambda i: (i, 0)),
            pl.BlockSpec((BN, IN), lambda i: (i, 0)),
            pl.BlockSpec((BN, IN), lambda i: (i, 0)),
            pl.BlockSpec((BN, 16), lambda i: (i, 0)),
            pl.BlockSpec((BN, 16), lambda i: (i, 0)),
            pl.BlockSpec((BN, 1), lambda i: (i, 0)),
            pl.BlockSpec((C, IN, H), lambda i: (0, 0, 0)),
            pl.BlockSpec((C, H), lambda i: (0, 0)),
        ],
        out_specs=[
            pl.BlockSpec((BN, H // 2), lambda i: (i, 0)),
            pl.BlockSpec((BN, H // 2), lambda i: (i, 0)),
        ],
        out_shape=[
            jax.ShapeDtypeStruct((N_SH, H // 2), jnp.float32),
            jax.ShapeDtypeStruct((N_SH, H // 2), jnp.float32),
        ],
    )(za, zb, y0, dega, degb, lab2d, W0, B0)


def _final_body(za_ref, zb_ref, ya_ref, yb_ref, dega_ref, degb_ref, lab_ref,
                bat_ref, w_ref, b_ref, wm1_ref, bm1_ref, gam_ref, bet_ref,
                wm2_ref, bm2_ref, out_ref, acc_ref):
    i = pl.program_id(0)

    @pl.when(i == 0)
    def _():
        acc_ref[...] = jnp.zeros((G, H), jnp.float32)

    dis = _dis_block(dega_ref, degb_ref)
    agg = jnp.concatenate(
        [za_ref[...] + ya_ref[...], zb_ref[...] + yb_ref[...]], axis=1) * dis
    lab = lab_ref[...]
    b = b_ref[...]
    x2 = jnp.zeros((BN, H), jnp.float32)
    for j in range(C):
        m = (lab == j).astype(jnp.float32)
        x2 = x2 + m * (jnp.dot(agg, w_ref[j],
                               preferred_element_type=jnp.float32) + b[j][None, :])
    onehot = (bat_ref[...] == lax.broadcasted_iota(jnp.int32, (1, G), 1)
              ).astype(jnp.float32)
    acc_ref[...] += lax.dot_general(
        onehot, x2, (((0,), (0,)), ((), ())),
        preferred_element_type=jnp.float32)

    @pl.when(i == GRID - 1)
    def _():
        pooled = acc_ref[...]
        hm = jnp.dot(pooled, wm1_ref[...],
                     preferred_element_type=jnp.float32) + bm1_ref[...]
        mu = jnp.mean(hm, axis=0, keepdims=True)
        var = jnp.mean((hm - mu) ** 2, axis=0, keepdims=True)
        hm = (hm - mu) * lax.rsqrt(var + 1e-5) * gam_ref[...] + bet_ref[...]
        hm = jnp.maximum(hm, 0.0)
        out_ref[...] = jnp.dot(hm, wm2_ref[...],
                               preferred_element_type=jnp.float32) + bm2_ref[...]


def _final_call(za, zb, ya, yb, dega, degb, lab2d, bat2d, W1, B1,
                Wm1, bm1, gamma, beta, Wm2, bm2):
    fc = H // 2
    return pl.pallas_call(
        _final_body,
        grid=(GRID,),
        in_specs=[
            pl.BlockSpec((BN, fc), lambda i: (i, 0)),
            pl.BlockSpec((BN, fc), lambda i: (i, 0)),
            pl.BlockSpec((BN, fc), lambda i: (i, 0)),
            pl.BlockSpec((BN, fc), lambda i: (i, 0)),
            pl.BlockSpec((BN, 16), lambda i: (i, 0)),
            pl.BlockSpec((BN, 16), lambda i: (i, 0)),
            pl.BlockSpec((BN, 1), lambda i: (i, 0)),
            pl.BlockSpec((BN, 1), lambda i: (i, 0)),
            pl.BlockSpec((C, H, H), lambda i: (0, 0, 0)),
            pl.BlockSpec((C, H), lambda i: (0, 0)),
            pl.BlockSpec((H, H), lambda i: (0, 0)),
            pl.BlockSpec((1, H), lambda i: (0, 0)),
            pl.BlockSpec((1, H), lambda i: (0, 0)),
            pl.BlockSpec((1, H), lambda i: (0, 0)),
            pl.BlockSpec((H, OUT), lambda i: (0, 0)),
            pl.BlockSpec((1, OUT), lambda i: (0, 0)),
        ],
        out_specs=pl.BlockSpec((G, OUT), lambda i: (0, 0)),
        out_shape=jax.ShapeDtypeStruct((G, OUT), jnp.float32),
        scratch_shapes=[pltpu.VMEM((G, H), jnp.float32)],
    )(za, zb, ya, yb, dega, degb, lab2d, bat2d, W1, B1,
      Wm1, bm1, gamma, beta, Wm2, bm2)


def kernel(x_feat, cluster_labels, edge_index, batch, W0, B0, W1, B1,
           Wm1, bm1, gamma, beta, Wm2, bm2):
    row = edge_index[0].astype(jnp.int32)
    col = edge_index[1].astype(jnp.int32)
    pad = E_PAD - E
    rowp = jnp.concatenate([row, jnp.full((pad,), N, jnp.int32)])
    colp = jnp.concatenate([col, jnp.full((pad,), N, jnp.int32)])
    row3d = rowp.reshape(NS, STEPS_AGG, B_EDGE)
    col3d = colp.reshape(NS, STEPS_AGG, B_EDGE)
    row3d_w = rowp.reshape(NC * NS, STEPS_DEG, B_EDGE)
    col3d_w = colp.reshape(NC * NS, STEPS_DEG, B_EDGE)

    lab2d = cluster_labels.astype(jnp.int32).reshape(N, 1)
    bat2d = batch.astype(jnp.int32).reshape(N, 1)

    dega, degb = _make_deg_kernel()(col3d_w)
    y0 = _scale_call(dega, degb, x_feat)
    z0a, z0b = _make_agg0_kernel()(row3d_w, col3d_w, y0)
    y1a, y1b = _layer0_call(z0a, z0b, y0, dega, degb, lab2d, W0, B0)
    z1a, z1b = _make_agg1_kernel()(row3d, col3d, y1a, y1b)
    return _final_call(z1a, z1b, y1a, y1b, dega, degb, lab2d, bat2d, W1, B1,
                       Wm1, bm1.reshape(1, H), gamma.reshape(1, H),
                       beta.reshape(1, H), Wm2, bm2.reshape(1, OUT))


# speedup vs baseline: 26.6721x; 1.1204x over previous
import functools

import jax
import jax.numpy as jnp
from jax import lax
from jax.experimental import pallas as pl
from jax.experimental.pallas import tpu as pltpu
from jax.experimental.pallas import tpu_sc as plsc

N = 10000
E = 320000
IN = 128
H = 256
OUT = 128
C = 4
G = 64

NC = 2
NS = 16
B_EDGE = 128
STEPS_AGG = 160
STEPS_DEG = 80
CH = 16
E_PAD = NS * STEPS_AGG * B_EDGE
N_SH = 10112
RPT = N_SH // NS

BN = 400
GRID = N // BN


def _zero_fill(buf, rows, width):
    def body(i, _):
        for k in range(width // 16):
            buf[i, pl.ds(16 * k, 16)] = jnp.zeros((16,), jnp.float32)
        return 0
    lax.fori_loop(0, rows, body, 0, unroll=False)


def _copy_zero_rows(buf, dst, base):
    off = 0
    for sz in (128, 128, 128, 128, RPT - 512):
        assert sz > 0
        pltpu.sync_copy(buf.at[pl.ds(0, sz)], dst.at[pl.ds(base + off, sz)])
        off += sz


def _pipelined_edge_loop(y_hbm, row3d, col3d, edge_base, steps, rowv, colv,
                         rbuf, shz, sem):
    n_chunks = steps // CH
    pltpu.sync_copy(row3d.at[edge_base, pl.ds(0, CH)], rowv[0])
    pltpu.sync_copy(col3d.at[edge_base, pl.ds(0, CH)], colv[0])
    h = pltpu.async_copy(y_hbm.at[rowv[0].at[0]], rbuf[0], sem)
    for j in range(steps):
        ci, jj = divmod(j, CH)
        nci, njj = divmod(j + 1, CH)
        if jj == CH - 1 and ci + 1 < n_chunks:
            pltpu.sync_copy(row3d.at[edge_base, pl.ds((ci + 1) * CH, CH)],
                            rowv[(ci + 1) % 2])
            pltpu.sync_copy(col3d.at[edge_base, pl.ds((ci + 1) * CH, CH)],
                            colv[(ci + 1) % 2])
        h.wait()
        if j + 1 < steps:
            h = pltpu.async_copy(y_hbm.at[rowv[nci % 2].at[njj]],
                                 rbuf[(j + 1) % 2], sem)
        pltpu.sync_copy(rbuf[j % 2], shz.at[colv[ci % 2].at[jj]], add=True)


def _make_deg_kernel():
    mesh = plsc.VectorSubcoreMesh(core_axis_name="c", subcore_axis_name="s")

    @functools.partial(
        pl.kernel, mesh=mesh,
        out_type=(jax.ShapeDtypeStruct((N_SH, 16), jnp.float32),
                  jax.ShapeDtypeStruct((N_SH, 16), jnp.float32)),
        scratch_types=[
            pltpu.VMEM((STEPS_DEG, B_EDGE), jnp.int32),
            pltpu.VMEM((B_EDGE, 16), jnp.float32),
            pltpu.VMEM_SHARED((N_SH, 16), jnp.float32),
        ],
    )
    def deg_kernel(col3d, dega, degb, colv, buf, shd):
        cid = lax.axis_index("c")
        sid = lax.axis_index("s")
        wid = cid * NS + sid
        pltpu.sync_copy(col3d.at[wid], colv)
        _zero_fill(buf, B_EDGE, 16)
        _copy_zero_rows(buf, shd, sid * RPT)

        def fill_ones(i, _):
            buf[i, pl.ds(0, 16)] = jnp.ones((16,), jnp.float32)
            return 0
        lax.fori_loop(0, B_EDGE, fill_ones, 0, unroll=False)

        plsc.subcore_barrier()

        def body(j, _):
            pltpu.sync_copy(buf, shd.at[colv.at[j]], add=True)
            return 0
        lax.fori_loop(0, STEPS_DEG, body, 0, unroll=False)

        plsc.subcore_barrier()

        @pl.when(cid == 0)
        def _():
            pltpu.sync_copy(shd.at[pl.ds(sid * RPT, RPT)],
                            dega.at[pl.ds(sid * RPT, RPT)])

        @pl.when(cid == 1)
        def _():
            pltpu.sync_copy(shd.at[pl.ds(sid * RPT, RPT)],
                            degb.at[pl.ds(sid * RPT, RPT)])

    return deg_kernel


def _make_agg0_kernel():
    mesh = plsc.VectorSubcoreMesh(core_axis_name="c", subcore_axis_name="s")

    @functools.partial(
        pl.kernel, mesh=mesh,
        out_type=(jax.ShapeDtypeStruct((N_SH, IN), jnp.float32),
                  jax.ShapeDtypeStruct((N_SH, IN), jnp.float32)),
        scratch_types=[
            pltpu.VMEM((CH, B_EDGE), jnp.int32),
            pltpu.VMEM((CH, B_EDGE), jnp.int32),
            pltpu.VMEM((CH, B_EDGE), jnp.int32),
            pltpu.VMEM((CH, B_EDGE), jnp.int32),
            pltpu.VMEM((B_EDGE, IN), jnp.float32),
            pltpu.VMEM((B_EDGE, IN), jnp.float32),
            pltpu.VMEM_SHARED((N_SH, IN), jnp.float32),
            pltpu.SemaphoreType.DMA,
        ],
    )
    def agg0_kernel(row3d, col3d, y0, za, zb, rowv0, rowv1, colv0, colv1,
                    rbuf0, rbuf1, shz, sem):
        cid = lax.axis_index("c")
        sid = lax.axis_index("s")
        wid = cid * NS + sid
        _zero_fill(rbuf0, B_EDGE, IN)
        _copy_zero_rows(rbuf0, shz, sid * RPT)
        plsc.subcore_barrier()

        _pipelined_edge_loop(y0, row3d, col3d, wid, STEPS_DEG,
                             (rowv0, rowv1), (colv0, colv1), (rbuf0, rbuf1),
                             shz, sem)

        plsc.subcore_barrier()

        @pl.when(cid == 0)
        def _():
            pltpu.sync_copy(shz.at[pl.ds(sid * RPT, RPT)],
                            za.at[pl.ds(sid * RPT, RPT)])

        @pl.when(cid == 1)
        def _():
            pltpu.sync_copy(shz.at[pl.ds(sid * RPT, RPT)],
                            zb.at[pl.ds(sid * RPT, RPT)])

    return agg0_kernel


def _make_agg1_kernel():
    fc = H // 2
    mesh = plsc.VectorSubcoreMesh(core_axis_name="c", subcore_axis_name="s")

    @functools.partial(
        pl.kernel, mesh=mesh,
        out_type=(jax.ShapeDtypeStruct((N_SH, fc), jnp.float32),
                  jax.ShapeDtypeStruct((N_SH, fc), jnp.float32)),
        scratch_types=[
            pltpu.VMEM((CH, B_EDGE), jnp.int32),
            pltpu.VMEM((CH, B_EDGE), jnp.int32),
            pltpu.VMEM((CH, B_EDGE), jnp.int32),
            pltpu.VMEM((CH, B_EDGE), jnp.int32),
            pltpu.VMEM((B_EDGE, fc), jnp.float32),
            pltpu.VMEM((B_EDGE, fc), jnp.float32),
            pltpu.VMEM_SHARED((N_SH, fc), jnp.float32),
            pltpu.SemaphoreType.DMA,
        ],
    )
    def agg1_kernel(row3d, col3d, ya, yb, za, zb, rowv0, rowv1, colv0, colv1,
                    rbuf0, rbuf1, shz, sem):
        cid = lax.axis_index("c")
        sid = lax.axis_index("s")
        _zero_fill(rbuf0, B_EDGE, fc)
        _copy_zero_rows(rbuf0, shz, sid * RPT)
        plsc.subcore_barrier()

        def run(y_hbm):
            _pipelined_edge_loop(y_hbm, row3d, col3d, sid, STEPS_AGG,
                                 (rowv0, rowv1), (colv0, colv1),
                                 (rbuf0, rbuf1), shz, sem)

        @pl.when(cid == 0)
        def _():
            run(ya)

        @pl.when(cid == 1)
        def _():
            run(yb)

        plsc.subcore_barrier()

        @pl.when(cid == 0)
        def _():
            pltpu.sync_copy(shz.at[pl.ds(sid * RPT, RPT)],
                            za.at[pl.ds(sid * RPT, RPT)])

        @pl.when(cid == 1)
        def _():
            pltpu.sync_copy(shz.at[pl.ds(sid * RPT, RPT)],
                            zb.at[pl.ds(sid * RPT, RPT)])

    return agg1_kernel


def _dis_block(dega_ref, degb_ref):
    deg = dega_ref[:, 0:1] + degb_ref[:, 0:1] + 1.0
    return lax.rsqrt(deg)


def _scale_body(dega_ref, degb_ref, x_ref, y_ref):
    dis = _dis_block(dega_ref, degb_ref)
    y_ref[...] = x_ref[...] * dis


def _scale_call(dega, degb, x_feat):
    return pl.pallas_call(
        _scale_body,
        grid=(GRID,),
        in_specs=[
            pl.BlockSpec((BN, 16), lambda i: (i, 0)),
            pl.BlockSpec((BN, 16), lambda i: (i, 0)),
            pl.BlockSpec((BN, IN), lambda i: (i, 0)),
        ],
        out_specs=pl.BlockSpec((BN, IN), lambda i: (i, 0)),
        out_shape=jax.ShapeDtypeStruct((N_SH, IN), jnp.float32),
    )(dega, degb, x_feat)


def _layer0_body(za_ref, zb_ref, y0_ref, dega_ref, degb_ref, lab_ref,
                 w_ref, b_ref, y1a_ref, y1b_ref):
    dis = _dis_block(dega_ref, degb_ref)
    agg = (za_ref[...] + zb_ref[...] + y0_ref[...]) * dis
    lab = lab_ref[...]
    b = b_ref[...]
    x1 = jnp.zeros((BN, H), jnp.float32)
    for j in range(C):
        m = (lab == j).astype(jnp.float32)
        x1 = x1 + m * (jnp.dot(agg, w_ref[j],
                               preferred_element_type=jnp.float32) + b[j][None, :])
    y1 = x1 * dis
    y1a_ref[...] = y1[:, :H // 2]
    y1b_ref[...] = y1[:, H // 2:]


def _layer0_call(za, zb, y0, dega, degb, lab2d, W0, B0):
    return pl.pallas_call(
        _layer0_body,
        grid=(GRID,),
        in_specs=[
            pl.BlockSpec((BN, IN), lambda i: (i, 0)),
            pl.BlockSpec((BN, IN), lambda i: (i, 0)),
            pl.BlockSpec((BN, IN), lambda i: (i, 0)),
            pl.BlockSpec((BN, 16), lambda i: (i, 0)),
            pl.BlockSpec((BN, 16), lambda i: (i, 0)),
            pl.BlockSpec((BN, 1), lambda i: (i, 0)),
            pl.BlockSpec((C, IN, H), lambda i: (0, 0, 0)),
            pl.BlockSpec((C, H), lambda i: (0, 0)),
        ],
        out_specs=[
            pl.BlockSpec((BN, H // 2), lambda i: (i, 0)),
            pl.BlockSpec((BN, H // 2), lambda i: (i, 0)),
        ],
        out_shape=[
            jax.ShapeDtypeStruct((N_SH, H // 2), jnp.float32),
            jax.ShapeDtypeStruct((N_SH, H // 2), jnp.float32),
        ],
    )(za, zb, y0, dega, degb, lab2d, W0, B0)


def _final_body(za_ref, zb_ref, ya_ref, yb_ref, dega_ref, degb_ref, lab_ref,
                bat_ref, w_ref, b_ref, wm1_ref, bm1_ref, gam_ref, bet_ref,
                wm2_ref, bm2_ref, out_ref, acc_ref):
    i = pl.program_id(0)

    @pl.when(i == 0)
    def _():
        acc_ref[...] = jnp.zeros((G, H), jnp.float32)

    dis = _dis_block(dega_ref, degb_ref)
    agg = jnp.concatenate(
        [za_ref[...] + ya_ref[...], zb_ref[...] + yb_ref[...]], axis=1) * dis
    lab = lab_ref[...]
    b = b_ref[...]
    x2 = jnp.zeros((BN, H), jnp.float32)
    for j in range(C):
        m = (lab == j).astype(jnp.float32)
        x2 = x2 + m * (jnp.dot(agg, w_ref[j],
                               preferred_element_type=jnp.float32) + b[j][None, :])
    onehot = (bat_ref[...] == lax.broadcasted_iota(jnp.int32, (1, G), 1)
              ).astype(jnp.float32)
    acc_ref[...] += lax.dot_general(
        onehot, x2, (((0,), (0,)), ((), ())),
        preferred_element_type=jnp.float32)

    @pl.when(i == GRID - 1)
    def _():
        pooled = acc_ref[...]
        hm = jnp.dot(pooled, wm1_ref[...],
                     preferred_element_type=jnp.float32) + bm1_ref[...]
        mu = jnp.mean(hm, axis=0, keepdims=True)
        var = jnp.mean((hm - mu) ** 2, axis=0, keepdims=True)
        hm = (hm - mu) * lax.rsqrt(var + 1e-5) * gam_ref[...] + bet_ref[...]
        hm = jnp.maximum(hm, 0.0)
        out_ref[...] = jnp.dot(hm, wm2_ref[...],
                               preferred_element_type=jnp.float32) + bm2_ref[...]


def _final_call(za, zb, ya, yb, dega, degb, lab2d, bat2d, W1, B1,
                Wm1, bm1, gamma, beta, Wm2, bm2):
    fc = H // 2
    return pl.pallas_call(
        _final_body,
        grid=(GRID,),
        in_specs=[
            pl.BlockSpec((BN, fc), lambda i: (i, 0)),
            pl.BlockSpec((BN, fc), lambda i: (i, 0)),
            pl.BlockSpec((BN, fc), lambda i: (i, 0)),
            pl.BlockSpec((BN, fc), lambda i: (i, 0)),
            pl.BlockSpec((BN, 16), lambda i: (i, 0)),
            pl.BlockSpec((BN, 16), lambda i: (i, 0)),
            pl.BlockSpec((BN, 1), lambda i: (i, 0)),
            pl.BlockSpec((BN, 1), lambda i: (i, 0)),
            pl.BlockSpec((C, H, H), lambda i: (0, 0, 0)),
            pl.BlockSpec((C, H), lambda i: (0, 0)),
            pl.BlockSpec((H, H), lambda i: (0, 0)),
            pl.BlockSpec((1, H), lambda i: (0, 0)),
            pl.BlockSpec((1, H), lambda i: (0, 0)),
            pl.BlockSpec((1, H), lambda i: (0, 0)),
            pl.BlockSpec((H, OUT), lambda i: (0, 0)),
            pl.BlockSpec((1, OUT), lambda i: (0, 0)),
        ],
        out_specs=pl.BlockSpec((G, OUT), lambda i: (0, 0)),
        out_shape=jax.ShapeDtypeStruct((G, OUT), jnp.float32),
        scratch_shapes=[pltpu.VMEM((G, H), jnp.float32)],
    )(za, zb, ya, yb, dega, degb, lab2d, bat2d, W1, B1,
      Wm1, bm1, gamma, beta, Wm2, bm2)


def kernel(x_feat, cluster_labels, edge_index, batch, W0, B0, W1, B1,
           Wm1, bm1, gamma, beta, Wm2, bm2):
    row = edge_index[0].astype(jnp.int32)
    col = edge_index[1].astype(jnp.int32)
    pad = E_PAD - E
    rowp = jnp.concatenate([row, jnp.full((pad,), N, jnp.int32)])
    colp = jnp.concatenate([col, jnp.full((pad,), N, jnp.int32)])
    row3d = rowp.reshape(NS, STEPS_AGG, B_EDGE)
    col3d = colp.reshape(NS, STEPS_AGG, B_EDGE)
    row3d_w = rowp.reshape(NC * NS, STEPS_DEG, B_EDGE)
    col3d_w = colp.reshape(NC * NS, STEPS_DEG, B_EDGE)

    lab2d = cluster_labels.astype(jnp.int32).reshape(N, 1)
    bat2d = batch.astype(jnp.int32).reshape(N, 1)

    dega, degb = _make_deg_kernel()(col3d_w)
    y0 = _scale_call(dega, degb, x_feat)
    z0a, z0b = _make_agg0_kernel()(row3d_w, col3d_w, y0)
    y1a, y1b = _layer0_call(z0a, z0b, y0, dega, degb, lab2d, W0, B0)
    z1a, z1b = _make_agg1_kernel()(row3d, col3d, y1a, y1b)
    return _final_call(z1a, z1b, y1a, y1b, dega, degb, lab2d, bat2d, W1, B1,
                       Wm1, bm1.reshape(1, H), gamma.reshape(1, H),
                       beta.reshape(1, H), Wm2, bm2.reshape(1, OUT))


# speedup vs baseline: 63.5050x; 2.3810x over previous
import functools

import jax
import jax.numpy as jnp
from jax import lax
from jax.experimental import pallas as pl
from jax.experimental.pallas import tpu as pltpu
from jax.experimental.pallas import tpu_sc as plsc

N = 10000
E = 320000
IN = 128
H = 256
OUT = 128
C = 4
G = 64

NC = 2
NS = 16
B_EDGE = 128
STEPS_AGG = 160
STEPS_DEG = 80
CH = 16
E_PAD = NS * STEPS_AGG * B_EDGE
N_SH = 10112
RPT = N_SH // NS

BN = 400
GRID = N // BN


def _zero_fill(buf, rows, width):
    def body(i, _):
        for k in range(width // 16):
            buf[i, pl.ds(16 * k, 16)] = jnp.zeros((16,), jnp.float32)
        return 0
    lax.fori_loop(0, rows, body, 0, unroll=False)


def _copy_zero_rows(buf, dst, base):
    off = 0
    for sz in (128, 128, 128, 128, RPT - 512):
        assert sz > 0
        pltpu.sync_copy(buf.at[pl.ds(0, sz)], dst.at[pl.ds(base + off, sz)])
        off += sz


def _pipelined_edge_loop(y_hbm, row3d, col3d, edge_base, steps, rowv, colv,
                         rbuf, shz, sem):
    n_chunks = steps // CH
    pltpu.sync_copy(row3d.at[edge_base, pl.ds(0, CH)], rowv[0])
    pltpu.sync_copy(col3d.at[edge_base, pl.ds(0, CH)], colv[0])
    h = pltpu.async_copy(y_hbm.at[rowv[0].at[0]], rbuf[0], sem)
    for j in range(steps):
        ci, jj = divmod(j, CH)
        nci, njj = divmod(j + 1, CH)
        if jj == CH - 1 and ci + 1 < n_chunks:
            pltpu.sync_copy(row3d.at[edge_base, pl.ds((ci + 1) * CH, CH)],
                            rowv[(ci + 1) % 2])
            pltpu.sync_copy(col3d.at[edge_base, pl.ds((ci + 1) * CH, CH)],
                            colv[(ci + 1) % 2])
        h.wait()
        if j + 1 < steps:
            h = pltpu.async_copy(y_hbm.at[rowv[nci % 2].at[njj]],
                                 rbuf[(j + 1) % 2], sem)
        pltpu.sync_copy(rbuf[j % 2], shz.at[colv[ci % 2].at[jj]], add=True)


def _make_deg_kernel():
    mesh = plsc.VectorSubcoreMesh(core_axis_name="c", subcore_axis_name="s")

    @functools.partial(
        pl.kernel, mesh=mesh,
        out_type=(jax.ShapeDtypeStruct((N_SH, 16), jnp.float32),
                  jax.ShapeDtypeStruct((N_SH, 16), jnp.float32)),
        scratch_types=[
            pltpu.VMEM((STEPS_DEG, B_EDGE), jnp.int32),
            pltpu.VMEM((B_EDGE, 16), jnp.float32),
            pltpu.VMEM_SHARED((N_SH, 16), jnp.float32),
        ],
    )
    def deg_kernel(col3d, dega, degb, colv, buf, shd):
        cid = lax.axis_index("c")
        sid = lax.axis_index("s")
        wid = cid * NS + sid
        pltpu.sync_copy(col3d.at[wid], colv)
        _zero_fill(buf, B_EDGE, 16)
        _copy_zero_rows(buf, shd, sid * RPT)

        def fill_ones(i, _):
            buf[i, pl.ds(0, 16)] = jnp.ones((16,), jnp.float32)
            return 0
        lax.fori_loop(0, B_EDGE, fill_ones, 0, unroll=False)

        plsc.subcore_barrier()

        def body(j, _):
            pltpu.sync_copy(buf, shd.at[colv.at[j]], add=True)
            return 0
        lax.fori_loop(0, STEPS_DEG, body, 0, unroll=False)

        plsc.subcore_barrier()

        @pl.when(cid == 0)
        def _():
            pltpu.sync_copy(shd.at[pl.ds(sid * RPT, RPT)],
                            dega.at[pl.ds(sid * RPT, RPT)])

        @pl.when(cid == 1)
        def _():
            pltpu.sync_copy(shd.at[pl.ds(sid * RPT, RPT)],
                            degb.at[pl.ds(sid * RPT, RPT)])

    return deg_kernel


def _make_agg0_kernel():
    mesh = plsc.VectorSubcoreMesh(core_axis_name="c", subcore_axis_name="s")

    @functools.partial(
        pl.kernel, mesh=mesh,
        out_type=(jax.ShapeDtypeStruct((N_SH, IN), jnp.float32),
                  jax.ShapeDtypeStruct((N_SH, IN), jnp.float32)),
        scratch_types=[
            pltpu.VMEM((CH, B_EDGE), jnp.int32),
            pltpu.VMEM((CH, B_EDGE), jnp.int32),
            pltpu.VMEM((CH, B_EDGE), jnp.int32),
            pltpu.VMEM((CH, B_EDGE), jnp.int32),
            pltpu.VMEM((B_EDGE, IN), jnp.float32),
            pltpu.VMEM((B_EDGE, IN), jnp.float32),
            pltpu.VMEM_SHARED((N_SH, IN), jnp.float32),
            pltpu.SemaphoreType.DMA,
        ],
    )
    def agg0_kernel(row3d, col3d, y0, za, zb, rowv0, rowv1, colv0, colv1,
                    rbuf0, rbuf1, shz, sem):
        cid = lax.axis_index("c")
        sid = lax.axis_index("s")
        wid = cid * NS + sid
        _zero_fill(rbuf0, B_EDGE, IN)
        _copy_zero_rows(rbuf0, shz, sid * RPT)
        plsc.subcore_barrier()

        _pipelined_edge_loop(y0, row3d, col3d, wid, STEPS_DEG,
                             (rowv0, rowv1), (colv0, colv1), (rbuf0, rbuf1),
                             shz, sem)

        plsc.subcore_barrier()

        @pl.when(cid == 0)
        def _():
            pltpu.sync_copy(shz.at[pl.ds(sid * RPT, RPT)],
                            za.at[pl.ds(sid * RPT, RPT)])

        @pl.when(cid == 1)
        def _():
            pltpu.sync_copy(shz.at[pl.ds(sid * RPT, RPT)],
                            zb.at[pl.ds(sid * RPT, RPT)])

    return agg0_kernel


def _make_agg1_kernel():
    fc = H // 2
    mesh = plsc.VectorSubcoreMesh(core_axis_name="c", subcore_axis_name="s")

    @functools.partial(
        pl.kernel, mesh=mesh,
        out_type=(jax.ShapeDtypeStruct((N_SH, fc), jnp.float32),
                  jax.ShapeDtypeStruct((N_SH, fc), jnp.float32)),
        scratch_types=[
            pltpu.VMEM((CH, B_EDGE), jnp.int32),
            pltpu.VMEM((CH, B_EDGE), jnp.int32),
            pltpu.VMEM((CH, B_EDGE), jnp.int32),
            pltpu.VMEM((CH, B_EDGE), jnp.int32),
            pltpu.VMEM((B_EDGE, fc), jnp.float32),
            pltpu.VMEM((B_EDGE, fc), jnp.float32),
            pltpu.VMEM_SHARED((N_SH, fc), jnp.float32),
            pltpu.SemaphoreType.DMA,
        ],
    )
    def agg1_kernel(row3d, col3d, ya, yb, za, zb, rowv0, rowv1, colv0, colv1,
                    rbuf0, rbuf1, shz, sem):
        cid = lax.axis_index("c")
        sid = lax.axis_index("s")
        _zero_fill(rbuf0, B_EDGE, fc)
        _copy_zero_rows(rbuf0, shz, sid * RPT)
        plsc.subcore_barrier()

        def run(y_hbm):
            _pipelined_edge_loop(y_hbm, row3d, col3d, sid, STEPS_AGG,
                                 (rowv0, rowv1), (colv0, colv1),
                                 (rbuf0, rbuf1), shz, sem)

        @pl.when(cid == 0)
        def _():
            run(ya)

        @pl.when(cid == 1)
        def _():
            run(yb)

        plsc.subcore_barrier()

        @pl.when(cid == 0)
        def _():
            pltpu.sync_copy(shz.at[pl.ds(sid * RPT, RPT)],
                            za.at[pl.ds(sid * RPT, RPT)])

        @pl.when(cid == 1)
        def _():
            pltpu.sync_copy(shz.at[pl.ds(sid * RPT, RPT)],
                            zb.at[pl.ds(sid * RPT, RPT)])

    return agg1_kernel


def _dis_block(dega_ref, degb_ref):
    deg = dega_ref[:, 0:1] + degb_ref[:, 0:1] + 1.0
    return lax.rsqrt(deg)


def _scale_body(dega_ref, degb_ref, x_ref, y_ref):
    dis = _dis_block(dega_ref, degb_ref)
    y_ref[...] = x_ref[...] * dis


def _scale_call(dega, degb, x_feat):
    return pl.pallas_call(
        _scale_body,
        grid=(GRID,),
        in_specs=[
            pl.BlockSpec((BN, 16), lambda i: (i, 0)),
            pl.BlockSpec((BN, 16), lambda i: (i, 0)),
            pl.BlockSpec((BN, IN), lambda i: (i, 0)),
        ],
        out_specs=pl.BlockSpec((BN, IN), lambda i: (i, 0)),
        out_shape=jax.ShapeDtypeStruct((N_SH, IN), jnp.float32),
    )(dega, degb, x_feat)


def _layer0_body(za_ref, zb_ref, y0_ref, dega_ref, degb_ref, lab_ref,
                 w_ref, b_ref, y1a_ref, y1b_ref):
    dis = _dis_block(dega_ref, degb_ref)
    agg = (za_ref[...] + zb_ref[...] + y0_ref[...]) * dis
    lab = lab_ref[...]
    b = b_ref[...]
    x1 = jnp.zeros((BN, H), jnp.float32)
    for j in range(C):
        m = (lab == j).astype(jnp.float32)
        x1 = x1 + m * (jnp.dot(agg, w_ref[j],
                               preferred_element_type=jnp.float32) + b[j][None, :])
    y1 = x1 * dis
    y1a_ref[...] = y1[:, :H // 2]
    y1b_ref[...] = y1[:, H // 2:]


def _layer0_call(za, zb, y0, dega, degb, lab2d, W0, B0):
    return pl.pallas_call(
        _layer0_body,
        grid=(GRID,),
        in_specs=[
            pl.BlockSpec((BN, IN), lambda i: (i, 0)),
            pl.BlockSpec((BN, IN), lambda i: (i, 0)),
            pl.BlockSpec((BN, IN), lambda i: (i, 0)),
            pl.BlockSpec((BN, 16), lambda i: (i, 0)),
            pl.BlockSpec((BN, 16), lambda i: (i, 0)),
            pl.BlockSpec((BN, 1), lambda i: (i, 0)),
            pl.BlockSpec((C, IN, H), lambda i: (0, 0, 0)),
            pl.BlockSpec((C, H), lambda i: (0, 0)),
        ],
        out_specs=[
            pl.BlockSpec((BN, H // 2), lambda i: (i, 0)),
            pl.BlockSpec((BN, H // 2), lambda i: (i, 0)),
        ],
        out_shape=[
            jax.ShapeDtypeStruct((N_SH, H // 2), jnp.float32),
            jax.ShapeDtypeStruct((N_SH, H // 2), jnp.float32),
        ],
    )(za, zb, y0, dega, degb, lab2d, W0, B0)


def _final_body(za_ref, zb_ref, ya_ref, yb_ref, dega_ref, degb_ref, lab_ref,
                bat_ref, w_ref, b_ref, wm1_ref, bm1_ref, gam_ref, bet_ref,
                wm2_ref, bm2_ref, out_ref, acc_ref):
    i = pl.program_id(0)

    @pl.when(i == 0)
    def _():
        acc_ref[...] = jnp.zeros((G, H), jnp.float32)

    dis = _dis_block(dega_ref, degb_ref)
    agg = jnp.concatenate(
        [za_ref[...] + ya_ref[...], zb_ref[...] + yb_ref[...]], axis=1) * dis
    lab = lab_ref[...]
    b = b_ref[...]
    x2 = jnp.zeros((BN, H), jnp.float32)
    for j in range(C):
        m = (lab == j).astype(jnp.float32)
        x2 = x2 + m * (jnp.dot(agg, w_ref[j],
                               preferred_element_type=jnp.float32) + b[j][None, :])
    onehot = (bat_ref[...] == lax.broadcasted_iota(jnp.int32, (1, G), 1)
              ).astype(jnp.float32)
    acc_ref[...] += lax.dot_general(
        onehot, x2, (((0,), (0,)), ((), ())),
        preferred_element_type=jnp.float32)

    @pl.when(i == GRID - 1)
    def _():
        pooled = acc_ref[...]
        hm = jnp.dot(pooled, wm1_ref[...],
                     preferred_element_type=jnp.float32) + bm1_ref[...]
        mu = jnp.mean(hm, axis=0, keepdims=True)
        var = jnp.mean((hm - mu) ** 2, axis=0, keepdims=True)
        hm = (hm - mu) * lax.rsqrt(var + 1e-5) * gam_ref[...] + bet_ref[...]
        hm = jnp.maximum(hm, 0.0)
        out_ref[...] = jnp.dot(hm, wm2_ref[...],
                               preferred_element_type=jnp.float32) + bm2_ref[...]


def _final_call(za, zb, ya, yb, dega, degb, lab2d, bat2d, W1, B1,
                Wm1, bm1, gamma, beta, Wm2, bm2):
    fc = H // 2
    return pl.pallas_call(
        _final_body,
        grid=(GRID,),
        in_specs=[
            pl.BlockSpec((BN, fc), lambda i: (i, 0)),
            pl.BlockSpec((BN, fc), lambda i: (i, 0)),
            pl.BlockSpec((BN, fc), lambda i: (i, 0)),
            pl.BlockSpec((BN, fc), lambda i: (i, 0)),
            pl.BlockSpec((BN, 16), lambda i: (i, 0)),
            pl.BlockSpec((BN, 16), lambda i: (i, 0)),
            pl.BlockSpec((BN, 1), lambda i: (i, 0)),
            pl.BlockSpec((BN, 1), lambda i: (i, 0)),
            pl.BlockSpec((C, H, H), lambda i: (0, 0, 0)),
            pl.BlockSpec((C, H), lambda i: (0, 0)),
            pl.BlockSpec((H, H), lambda i: (0, 0)),
            pl.BlockSpec((1, H), lambda i: (0, 0)),
            pl.BlockSpec((1, H), lambda i: (0, 0)),
            pl.BlockSpec((1, H), lambda i: (0, 0)),
            pl.BlockSpec((H, OUT), lambda i: (0, 0)),
            pl.BlockSpec((1, OUT), lambda i: (0, 0)),
        ],
        out_specs=pl.BlockSpec((G, OUT), lambda i: (0, 0)),
        out_shape=jax.ShapeDtypeStruct((G, OUT), jnp.float32),
        scratch_shapes=[pltpu.VMEM((G, H), jnp.float32)],
    )(za, zb, ya, yb, dega, degb, lab2d, bat2d, W1, B1,
      Wm1, bm1, gamma, beta, Wm2, bm2)


def kernel(x_feat, cluster_labels, edge_index, batch, W0, B0, W1, B1,
           Wm1, bm1, gamma, beta, Wm2, bm2):
    row = edge_index[0].astype(jnp.int32)
    col = edge_index[1].astype(jnp.int32)
    pad = E_PAD - E
    spread = N + (jnp.arange(pad, dtype=jnp.int32) % (N_SH - N))
    rowp = jnp.concatenate([row, spread])
    colp = jnp.concatenate([col, spread])
    row3d = rowp.reshape(NS, STEPS_AGG, B_EDGE)
    col3d = colp.reshape(NS, STEPS_AGG, B_EDGE)
    row3d_w = rowp.reshape(NC * NS, STEPS_DEG, B_EDGE)
    col3d_w = colp.reshape(NC * NS, STEPS_DEG, B_EDGE)

    lab2d = cluster_labels.astype(jnp.int32).reshape(N, 1)
    bat2d = batch.astype(jnp.int32).reshape(N, 1)

    dega, degb = _make_deg_kernel()(col3d_w)
    y0 = _scale_call(dega, degb, x_feat)
    z0a, z0b = _make_agg0_kernel()(row3d_w, col3d_w, y0)
    y1a, y1b = _layer0_call(z0a, z0b, y0, dega, degb, lab2d, W0, B0)
    z1a, z1b = _make_agg1_kernel()(row3d, col3d, y1a, y1b)
    return _final_call(z1a, z1b, y1a, y1b, dega, degb, lab2d, bat2d, W1, B1,
                       Wm1, bm1.reshape(1, H), gamma.reshape(1, H),
                       beta.reshape(1, H), Wm2, bm2.reshape(1, OUT))
